# Initial kernel scaffold; baseline (speedup 1.0000x reference)
import jax
import jax.numpy as jnp
from jax.experimental import pallas as pl


def kernel(input_ids, type_ids, dpe_ids, W_input, W_type, W_dpe, gamma, beta):
    raise NotImplementedError("write your pallas kernel here")


# speedup vs baseline: 4.9778x; 4.9778x over previous
import functools

import jax
import jax.numpy as jnp
from jax import lax
from jax.experimental import pallas as pl
from jax.experimental.pallas import tpu as pltpu
from jax.experimental.pallas import tpu_sc as plsc

_B, _S, _D = 4096, 200, 128
_N = _B * _S
_EPS = 1e-12
_V_TYPE, _V_DPE = 14, 25

_NC, _NS = 2, 16
_NW = _NC * _NS
_RPW = _N // _NW
_CHUNK = 128
_NCHUNK = _RPW // _CHUNK


def _sc_gather_sum(ids, ct, w_in, w_td):
    mesh = plsc.VectorSubcoreMesh(core_axis_name="c", subcore_axis_name="s")

    @functools.partial(
        pl.kernel,
        out_type=jax.ShapeDtypeStruct((_N, _D), jnp.float32),
        mesh=mesh,
        scratch_types=[
            pltpu.VMEM((_CHUNK,), jnp.int32),
            pltpu.VMEM((_CHUNK,), jnp.int32),
            pltpu.VMEM((_CHUNK, _D), jnp.float32),
            pltpu.VMEM((_CHUNK, _D), jnp.float32),
            pltpu.SemaphoreType.DMA,
        ],
    )
    def k(ids_hbm, ct_hbm, win_hbm, wtd_hbm, out_hbm, idx_v, ct_v, a_v, b_v, sem):
        wid = lax.axis_index("s") * _NC + lax.axis_index("c")
        base = wid * _RPW

        def chunk_body(c, carry):
            off = base + c * _CHUNK
            pltpu.sync_copy(ids_hbm.at[pl.ds(off, _CHUNK)], idx_v)
            pltpu.sync_copy(ct_hbm.at[pl.ds(off, _CHUNK)], ct_v)
            pltpu.async_copy(win_hbm.at[idx_v], a_v, sem).wait()
            pltpu.async_copy(wtd_hbm.at[ct_v], b_v, sem).wait()

            def add_body(i, c2):
                for j in range(_D // 16):
                    sl = pl.ds(j * 16, 16)
                    a_v[i, sl] += b_v[i, sl]
                return c2

            lax.fori_loop(0, _CHUNK, add_body, 0)
            pltpu.sync_copy(a_v, out_hbm.at[pl.ds(off, _CHUNK)])
            return carry

        lax.fori_loop(0, _NCHUNK, chunk_body, 0)

    return k(ids, ct, w_in, w_td)


_RBLK = 1024


def _tc_layernorm(x, gamma, beta):
    def body(x_ref, g_ref, b_ref, o_ref):
        xv = x_ref[...]
        mean = jnp.mean(xv, axis=1, keepdims=True)
        xc = xv - mean
        var = jnp.mean(xc * xc, axis=1, keepdims=True)
        o_ref[...] = xc * lax.rsqrt(var + _EPS) * g_ref[...] + b_ref[...]

    return pl.pallas_call(
        body,
        grid=(_N // _RBLK,),
        in_specs=[
            pl.BlockSpec((_RBLK, _D), lambda i: (i, 0)),
            pl.BlockSpec((1, _D), lambda i: (0, 0)),
            pl.BlockSpec((1, _D), lambda i: (0, 0)),
        ],
        out_specs=pl.BlockSpec((_RBLK, _D), lambda i: (i, 0)),
        out_shape=jax.ShapeDtypeStruct((_N, _D), jnp.float32),
    )(x, gamma.reshape(1, _D), beta.reshape(1, _D))


def kernel(input_ids, type_ids, dpe_ids, W_input, W_type, W_dpe, gamma, beta):
    ids = input_ids.reshape(_N).astype(jnp.int32)
    ct = (type_ids.reshape(_N).astype(jnp.int32) * _V_DPE
          + dpe_ids.reshape(_N).astype(jnp.int32))
    w_td = (W_type[:, None, :] + W_dpe[None, :, :]).reshape(_V_TYPE * _V_DPE, _D)
    s = _sc_gather_sum(ids, ct, W_input, w_td)
    y = _tc_layernorm(s, gamma, beta)
    return y.reshape(_B, _S, _D)
